# baseline (speedup 1.0000x reference)
import functools

import jax
import jax.numpy as jnp
from jax import lax
from jax.experimental import pallas as pl
from jax.experimental.pallas import tpu as pltpu
from jax.experimental.pallas import tpu_sc as plsc

_ROWS, _COLS = 128, 32768
_N = _ROWS * _COLS
_K = int(_N * 0.9)
_MIN32 = -2147483648

_NTILES = 32
_SHARD = _N // _NTILES
_CHUNK = 16384
_NCHUNK = _SHARD // _CHUNK
_LANES = 16


def _sortable_key(v):
    return jnp.where(v >= 0, v ^ _MIN32, ~v)


def _hist_body(nbins, bucket_shift, prefix_shift, x_hbm, prefix_hbm,
               out_hbm, buf_v, pbuf_v, hist_v, red_v):
    wid = lax.axis_index("c") * 16 + lax.axis_index("s")
    base = wid * _SHARD

    def zero(z, _):
        hist_v[pl.ds(z * _LANES, _LANES)] = jnp.zeros((_LANES,), jnp.int32)
        return 0

    lax.fori_loop(0, nbins, zero, 0, unroll=False)

    if prefix_shift is not None:
        pltpu.sync_copy(prefix_hbm, pbuf_v)
        pvec = pbuf_v[...]
    else:
        pvec = None
    liota = lax.iota(jnp.int32, _LANES)
    lane_base = liota * nbins
    ones = jnp.full((_LANES,), 1, jnp.int32)

    def chunk(ci, _):
        pltpu.sync_copy(x_hbm.at[pl.ds(base + ci * _CHUNK, _CHUNK)], buf_v)

        def step(j, _):
            key = _sortable_key(buf_v[pl.ds(j * _LANES, _LANES)])
            bucket = lax.shift_right_logical(key, bucket_shift) & (nbins - 1)
            idx = lane_base + bucket
            if prefix_shift is None:
                plsc.addupdate_scatter(hist_v, [idx], ones)
            else:
                m = lax.shift_right_logical(key, prefix_shift) == pvec
                plsc.addupdate_scatter(hist_v, [idx], ones, mask=m)
            return 0

        lax.fori_loop(0, _CHUNK // _LANES, step, 0, unroll=False)
        return 0

    lax.fori_loop(0, _NCHUNK, chunk, 0, unroll=False)

    def reduce_bins(c, _):
        acc = hist_v[pl.ds(c * _LANES, _LANES)]
        for l in range(1, _LANES):
            acc = acc + hist_v[pl.ds(l * nbins + c * _LANES, _LANES)]
        red_v[pl.ds(c * _LANES, _LANES)] = acc
        return 0

    lax.fori_loop(0, nbins // _LANES, reduce_bins, 0, unroll=False)
    pltpu.sync_copy(red_v, out_hbm.at[pl.ds(wid * nbins, nbins)])


def _make_hist_kernel(nbins, bucket_shift, prefix_shift):
    mesh = plsc.VectorSubcoreMesh(core_axis_name="c", subcore_axis_name="s")
    return functools.partial(
        pl.kernel,
        out_type=jax.ShapeDtypeStruct((_NTILES * nbins,), jnp.int32),
        mesh=mesh,
        scratch_types=[
            pltpu.VMEM((_CHUNK,), jnp.int32),
            pltpu.VMEM((_LANES,), jnp.int32),
            pltpu.VMEM((nbins * _LANES,), jnp.int32),
            pltpu.VMEM((nbins,), jnp.int32),
        ],
        compiler_params=pltpu.CompilerParams(needs_layout_passes=False),
    )(functools.partial(_hist_body, nbins, bucket_shift, prefix_shift))


_hist_p1 = _make_hist_kernel(4096, 20, None)
_hist_p2 = _make_hist_kernel(4096, 8, 20)
_hist_p3 = _make_hist_kernel(256, 0, 8)


def _pick_bin(tile_hists, nbins, k):
    hist = tile_hists.reshape(_NTILES, nbins).sum(axis=0)
    suffix = jnp.cumsum(hist[::-1])[::-1]
    b = jnp.sum((suffix >= k).astype(jnp.int32)) - 1
    k_next = k - (suffix[b] - hist[b])
    return b, k_next


def _mask_body(t_ref, x_ref, out_ref):
    ts = t_ref[0, 0] ^ _MIN32
    ubits = jnp.where(ts >= 0, ts, _MIN32 | (~ts))
    t = lax.bitcast_convert_type(ubits, jnp.float32)
    xv = x_ref[...]
    m = (xv >= t).astype(jnp.float32)
    out_ref[...] = xv + (m - xv)


def kernel(x):
    xi = lax.bitcast_convert_type(x, jnp.int32).reshape(-1)
    zeros16 = jnp.zeros((_LANES,), jnp.int32)

    h1 = _hist_p1(xi, zeros16)
    b1, k2 = _pick_bin(h1, 4096, _K)
    h2 = _hist_p2(xi, zeros16 + b1)
    b2, k3 = _pick_bin(h2, 4096, k2)
    h3 = _hist_p3(xi, zeros16 + ((b1 << 12) | b2))
    b3, _ = _pick_bin(h3, 256, k3)

    t_key = ((b1 << 20) | (b2 << 8) | b3).astype(jnp.int32).reshape(1, 1)

    block_rows = 8
    grid = _ROWS // block_rows
    out = pl.pallas_call(
        _mask_body,
        grid=(grid,),
        in_specs=[
            pl.BlockSpec(memory_space=pltpu.SMEM),
            pl.BlockSpec((block_rows, _COLS), lambda i: (i, 0)),
        ],
        out_specs=pl.BlockSpec((block_rows, _COLS), lambda i: (i, 0)),
        out_shape=jax.ShapeDtypeStruct((_ROWS, _COLS), jnp.float32),
    )(t_key, x)
    return out


# speedup vs baseline: 1.1419x; 1.1419x over previous
import functools

import jax
import jax.numpy as jnp
from jax import lax
from jax.experimental import pallas as pl
from jax.experimental.pallas import tpu as pltpu
from jax.experimental.pallas import tpu_sc as plsc

_ROWS, _COLS = 128, 32768
_N = _ROWS * _COLS
_K = int(_N * 0.9)
_MIN32 = -2147483648

_NTILES = 32
_SHARD = _N // _NTILES
_CHUNK = 16384
_NCHUNK = _SHARD // _CHUNK
_LANES = 16


def _sortable_key(v):
    return jnp.where(v >= 0, v ^ _MIN32, ~v)


def _hist_body(nbins, bucket_shift, prefix_shift, x_hbm, prefix_hbm,
               out_hbm, buf0_v, buf1_v, pbuf_v, hist_v, red_v, sem0, sem1):
    wid = lax.axis_index("c") * 16 + lax.axis_index("s")
    base = wid * _SHARD

    def zero(z, _):
        hist_v[pl.ds(z * _LANES, _LANES)] = jnp.zeros((_LANES,), jnp.int32)
        return 0

    lax.fori_loop(0, nbins, zero, 0, unroll=False)

    if prefix_shift is not None:
        pltpu.sync_copy(prefix_hbm, pbuf_v)
        pvec = pbuf_v[...]
    else:
        pvec = None
    liota = lax.iota(jnp.int32, _LANES)
    lane_base = liota * nbins
    ones = jnp.full((_LANES,), 1, jnp.int32)

    bufs = (buf0_v, buf1_v)
    sems = (sem0, sem1)
    descs = [None, None]
    descs[0] = pltpu.async_copy(x_hbm.at[pl.ds(base, _CHUNK)], bufs[0], sems[0])
    for ci in range(_NCHUNK):
        descs[ci % 2].wait()
        if ci + 1 < _NCHUNK:
            nxt = (ci + 1) % 2
            descs[nxt] = pltpu.async_copy(
                x_hbm.at[pl.ds(base + (ci + 1) * _CHUNK, _CHUNK)],
                bufs[nxt], sems[nxt])
        buf = bufs[ci % 2]

        def step(j, _):
            key = _sortable_key(
                plsc.bitcast(buf[pl.ds(j * _LANES, _LANES)], jnp.int32))
            bucket = lax.shift_right_logical(key, bucket_shift) & (nbins - 1)
            idx = lane_base + bucket
            if prefix_shift is None:
                plsc.addupdate_scatter(hist_v, [idx], ones)
            else:
                m = lax.shift_right_logical(key, prefix_shift) == pvec
                plsc.addupdate_scatter(hist_v, [idx], ones, mask=m)
            return 0

        lax.fori_loop(0, _CHUNK // _LANES, step, 0, unroll=8)

    def reduce_bins(c, _):
        acc = hist_v[pl.ds(c * _LANES, _LANES)]
        for l in range(1, _LANES):
            acc = acc + hist_v[pl.ds(l * nbins + c * _LANES, _LANES)]
        red_v[pl.ds(c * _LANES, _LANES)] = acc
        return 0

    lax.fori_loop(0, nbins // _LANES, reduce_bins, 0, unroll=False)
    pltpu.sync_copy(red_v, out_hbm.at[pl.ds(wid * nbins, nbins)])


def _make_hist_kernel(nbins, bucket_shift, prefix_shift):
    mesh = plsc.VectorSubcoreMesh(core_axis_name="c", subcore_axis_name="s")
    return functools.partial(
        pl.kernel,
        out_type=jax.ShapeDtypeStruct((_NTILES * nbins,), jnp.int32),
        mesh=mesh,
        scratch_types=[
            pltpu.VMEM((_CHUNK,), jnp.float32),
            pltpu.VMEM((_CHUNK,), jnp.float32),
            pltpu.VMEM((_LANES,), jnp.int32),
            pltpu.VMEM((nbins * _LANES,), jnp.int32),
            pltpu.VMEM((nbins,), jnp.int32),
            pltpu.SemaphoreType.DMA,
            pltpu.SemaphoreType.DMA,
        ],
        compiler_params=pltpu.CompilerParams(needs_layout_passes=False),
    )(functools.partial(_hist_body, nbins, bucket_shift, prefix_shift))


_hist_p1 = _make_hist_kernel(4096, 20, None)
_hist_p2 = _make_hist_kernel(4096, 8, 20)
_hist_p3 = _make_hist_kernel(256, 0, 8)


def _pick_bin(tile_hists, nbins, k):
    hist = tile_hists.reshape(_NTILES, nbins).sum(axis=0)
    suffix = jnp.cumsum(hist[::-1])[::-1]
    b = jnp.sum((suffix >= k).astype(jnp.int32)) - 1
    k_next = k - (suffix[b] - hist[b])
    return b, k_next


def _mask_body(t_ref, x_ref, out_ref):
    ts = t_ref[0, 0] ^ _MIN32
    ubits = jnp.where(ts >= 0, ts, _MIN32 | (~ts))
    t = lax.bitcast_convert_type(ubits, jnp.float32)
    xv = x_ref[...]
    m = (xv >= t).astype(jnp.float32)
    out_ref[...] = xv + (m - xv)


def kernel(x):
    xi = x.reshape(-1)
    zeros16 = jnp.zeros((_LANES,), jnp.int32)

    h1 = _hist_p1(xi, zeros16)
    b1, k2 = _pick_bin(h1, 4096, _K)
    h2 = _hist_p2(xi, zeros16 + b1)
    b2, k3 = _pick_bin(h2, 4096, k2)
    h3 = _hist_p3(xi, zeros16 + ((b1 << 12) | b2))
    b3, _ = _pick_bin(h3, 256, k3)

    t_key = ((b1 << 20) | (b2 << 8) | b3).astype(jnp.int32).reshape(1, 1)

    block_rows = 8
    grid = _ROWS // block_rows
    out = pl.pallas_call(
        _mask_body,
        grid=(grid,),
        in_specs=[
            pl.BlockSpec(memory_space=pltpu.SMEM),
            pl.BlockSpec((block_rows, _COLS), lambda i: (i, 0)),
        ],
        out_specs=pl.BlockSpec((block_rows, _COLS), lambda i: (i, 0)),
        out_shape=jax.ShapeDtypeStruct((_ROWS, _COLS), jnp.float32),
    )(t_key, x)
    return out


# speedup vs baseline: 3.2831x; 2.8751x over previous
import functools

import jax
import jax.numpy as jnp
from jax import lax
from jax.experimental import pallas as pl
from jax.experimental.pallas import tpu as pltpu
from jax.experimental.pallas import tpu_sc as plsc

_ROWS, _COLS = 128, 32768
_N = _ROWS * _COLS
_K = int(_N * 0.9)
_MIN32 = -2147483648

_NTILES = 32
_SHARD = _N // _NTILES
_CHUNK = 16384
_NCHUNK = _SHARD // _CHUNK
_LANES = 16


def _sortable_key(v):
    return jnp.where(v >= 0, v ^ _MIN32, ~v)


def _hist_body(nbins, bucket_shift, prefix_shift, x_hbm, prefix_hbm,
               out_hbm, buf0_v, buf1_v, pbuf_v, hist_v, red_v, sem0, sem1):
    wid = lax.axis_index("c") * 16 + lax.axis_index("s")
    base = wid * _SHARD

    @plsc.parallel_loop(0, nbins, unroll=8)
    def _(z):
        hist_v[pl.ds(z * _LANES, _LANES)] = jnp.zeros((_LANES,), jnp.int32)

    if prefix_shift is not None:
        pltpu.sync_copy(prefix_hbm, pbuf_v)
        pvec = pbuf_v[...]
    else:
        pvec = None
    liota = lax.iota(jnp.int32, _LANES)
    lane_base = liota * nbins
    ones = jnp.full((_LANES,), 1, jnp.int32)

    bufs = (buf0_v, buf1_v)
    sems = (sem0, sem1)
    descs = [None, None]
    descs[0] = pltpu.async_copy(x_hbm.at[pl.ds(base, _CHUNK)], bufs[0], sems[0])
    for ci in range(_NCHUNK):
        descs[ci % 2].wait()
        if ci + 1 < _NCHUNK:
            nxt = (ci + 1) % 2
            descs[nxt] = pltpu.async_copy(
                x_hbm.at[pl.ds(base + (ci + 1) * _CHUNK, _CHUNK)],
                bufs[nxt], sems[nxt])
        buf = bufs[ci % 2]

        @plsc.parallel_loop(0, _CHUNK // _LANES, unroll=8)
        def _(j):
            key = _sortable_key(
                plsc.bitcast(buf[pl.ds(j * _LANES, _LANES)], jnp.int32))
            bucket = lax.shift_right_logical(key, bucket_shift) & (nbins - 1)
            idx = lane_base + bucket
            if prefix_shift is None:
                plsc.addupdate_scatter(hist_v, [idx], ones)
            else:
                m = lax.shift_right_logical(key, prefix_shift) == pvec
                plsc.addupdate_scatter(hist_v, [idx], ones, mask=m)

    @plsc.parallel_loop(0, nbins // _LANES, unroll=4)
    def _(c):
        acc = hist_v[pl.ds(c * _LANES, _LANES)]
        for l in range(1, _LANES):
            acc = acc + hist_v[pl.ds(l * nbins + c * _LANES, _LANES)]
        red_v[pl.ds(c * _LANES, _LANES)] = acc
    pltpu.sync_copy(red_v, out_hbm.at[pl.ds(wid * nbins, nbins)])


def _make_hist_kernel(nbins, bucket_shift, prefix_shift):
    mesh = plsc.VectorSubcoreMesh(core_axis_name="c", subcore_axis_name="s")
    return functools.partial(
        pl.kernel,
        out_type=jax.ShapeDtypeStruct((_NTILES * nbins,), jnp.int32),
        mesh=mesh,
        scratch_types=[
            pltpu.VMEM((_CHUNK,), jnp.float32),
            pltpu.VMEM((_CHUNK,), jnp.float32),
            pltpu.VMEM((_LANES,), jnp.int32),
            pltpu.VMEM((nbins * _LANES,), jnp.int32),
            pltpu.VMEM((nbins,), jnp.int32),
            pltpu.SemaphoreType.DMA,
            pltpu.SemaphoreType.DMA,
        ],
        compiler_params=pltpu.CompilerParams(needs_layout_passes=False),
    )(functools.partial(_hist_body, nbins, bucket_shift, prefix_shift))


_hist_p1 = _make_hist_kernel(4096, 20, None)
_hist_p2 = _make_hist_kernel(4096, 8, 20)
_hist_p3 = _make_hist_kernel(256, 0, 8)


def _pick_bin(tile_hists, nbins, k):
    hist = tile_hists.reshape(_NTILES, nbins).sum(axis=0)
    suffix = jnp.cumsum(hist[::-1])[::-1]
    b = jnp.sum((suffix >= k).astype(jnp.int32)) - 1
    k_next = k - (suffix[b] - hist[b])
    return b, k_next


def _mask_body(t_ref, x_ref, out_ref):
    ts = t_ref[0, 0] ^ _MIN32
    ubits = jnp.where(ts >= 0, ts, _MIN32 | (~ts))
    t = lax.bitcast_convert_type(ubits, jnp.float32)
    xv = x_ref[...]
    m = (xv >= t).astype(jnp.float32)
    out_ref[...] = xv + (m - xv)


def kernel(x):
    xi = x.reshape(-1)
    zeros16 = jnp.zeros((_LANES,), jnp.int32)

    h1 = _hist_p1(xi, zeros16)
    b1, k2 = _pick_bin(h1, 4096, _K)
    h2 = _hist_p2(xi, zeros16 + b1)
    b2, k3 = _pick_bin(h2, 4096, k2)
    h3 = _hist_p3(xi, zeros16 + ((b1 << 12) | b2))
    b3, _ = _pick_bin(h3, 256, k3)

    t_key = ((b1 << 20) | (b2 << 8) | b3).astype(jnp.int32).reshape(1, 1)

    block_rows = 8
    grid = _ROWS // block_rows
    out = pl.pallas_call(
        _mask_body,
        grid=(grid,),
        in_specs=[
            pl.BlockSpec(memory_space=pltpu.SMEM),
            pl.BlockSpec((block_rows, _COLS), lambda i: (i, 0)),
        ],
        out_specs=pl.BlockSpec((block_rows, _COLS), lambda i: (i, 0)),
        out_shape=jax.ShapeDtypeStruct((_ROWS, _COLS), jnp.float32),
    )(t_key, x)
    return out


# speedup vs baseline: 3.8025x; 1.1582x over previous
import functools

import jax
import jax.numpy as jnp
from jax import lax
from jax.experimental import pallas as pl
from jax.experimental.pallas import tpu as pltpu
from jax.experimental.pallas import tpu_sc as plsc

_ROWS, _COLS = 128, 32768
_N = _ROWS * _COLS
_K = int(_N * 0.9)
_MIN32 = -2147483648

_NTILES = 32
_SHARD = _N // _NTILES
_CHUNK = 16384
_NCHUNK = _SHARD // _CHUNK
_LANES = 16


def _sortable_key(v):
    return jnp.where(v >= 0, v ^ _MIN32, ~v)


_ROWS_PER_TILE = _ROWS // _NTILES
_CHUNKS_PER_ROW = _COLS // _CHUNK


def _hist_body(nbins, bucket_shift, prefix_shift, x_hbm, prefix_hbm,
               out_hbm, buf0_v, buf1_v, pbuf_v, hist_v, red_v, sem0, sem1):
    wid = lax.axis_index("c") * 16 + lax.axis_index("s")
    row0 = wid * _ROWS_PER_TILE

    @plsc.parallel_loop(0, nbins, unroll=8)
    def _(z):
        hist_v[pl.ds(z * _LANES, _LANES)] = jnp.zeros((_LANES,), jnp.int32)

    if prefix_shift is not None:
        pltpu.sync_copy(prefix_hbm, pbuf_v)
        pvec = pbuf_v[...]
    else:
        pvec = None
    liota = lax.iota(jnp.int32, _LANES)
    lane_base = liota * nbins
    ones = jnp.full((_LANES,), 1, jnp.int32)

    bufs = (buf0_v, buf1_v)
    sems = (sem0, sem1)

    def _chunk_src(ci):
        row = row0 + ci // _CHUNKS_PER_ROW
        col = (ci % _CHUNKS_PER_ROW) * _CHUNK
        return x_hbm.at[row, pl.ds(col, _CHUNK)]

    descs = [None, None]
    descs[0] = pltpu.async_copy(_chunk_src(0), bufs[0], sems[0])
    for ci in range(_NCHUNK):
        descs[ci % 2].wait()
        if ci + 1 < _NCHUNK:
            nxt = (ci + 1) % 2
            descs[nxt] = pltpu.async_copy(_chunk_src(ci + 1), bufs[nxt], sems[nxt])
        buf = bufs[ci % 2]

        @plsc.parallel_loop(0, _CHUNK // _LANES, unroll=8)
        def _(j):
            key = _sortable_key(
                plsc.bitcast(buf[pl.ds(j * _LANES, _LANES)], jnp.int32))
            bucket = lax.shift_right_logical(key, bucket_shift) & (nbins - 1)
            idx = lane_base + bucket
            if prefix_shift is None:
                plsc.addupdate_scatter(hist_v, [idx], ones)
            else:
                m = lax.shift_right_logical(key, prefix_shift) == pvec
                plsc.addupdate_scatter(hist_v, [idx], ones, mask=m)

    @plsc.parallel_loop(0, nbins // _LANES, unroll=4)
    def _(c):
        acc = hist_v[pl.ds(c * _LANES, _LANES)]
        for l in range(1, _LANES):
            acc = acc + hist_v[pl.ds(l * nbins + c * _LANES, _LANES)]
        red_v[pl.ds(c * _LANES, _LANES)] = acc
    pltpu.sync_copy(red_v, out_hbm.at[pl.ds(wid * nbins, nbins)])


def _make_hist_kernel(nbins, bucket_shift, prefix_shift):
    mesh = plsc.VectorSubcoreMesh(core_axis_name="c", subcore_axis_name="s")
    return functools.partial(
        pl.kernel,
        out_type=jax.ShapeDtypeStruct((_NTILES * nbins,), jnp.int32),
        mesh=mesh,
        scratch_types=[
            pltpu.VMEM((_CHUNK,), jnp.float32),
            pltpu.VMEM((_CHUNK,), jnp.float32),
            pltpu.VMEM((_LANES,), jnp.int32),
            pltpu.VMEM((nbins * _LANES,), jnp.int32),
            pltpu.VMEM((nbins,), jnp.int32),
            pltpu.SemaphoreType.DMA,
            pltpu.SemaphoreType.DMA,
        ],
        compiler_params=pltpu.CompilerParams(needs_layout_passes=False),
    )(functools.partial(_hist_body, nbins, bucket_shift, prefix_shift))


_hist_p1 = _make_hist_kernel(4096, 20, None)
_hist_p2 = _make_hist_kernel(4096, 8, 20)
_hist_p3 = _make_hist_kernel(256, 0, 8)


def _pick_bin(tile_hists, nbins, k):
    hist = tile_hists.reshape(_NTILES, nbins).sum(axis=0)
    suffix = jnp.cumsum(hist[::-1])[::-1]
    b = jnp.sum((suffix >= k).astype(jnp.int32)) - 1
    k_next = k - (suffix[b] - hist[b])
    return b, k_next


def _mask_body(t_ref, x_ref, out_ref):
    ts = t_ref[0, 0] ^ _MIN32
    ubits = jnp.where(ts >= 0, ts, _MIN32 | (~ts))
    t = lax.bitcast_convert_type(ubits, jnp.float32)
    xv = x_ref[...]
    m = (xv >= t).astype(jnp.float32)
    out_ref[...] = xv + (m - xv)


def kernel(x):
    zeros16 = jnp.zeros((_LANES,), jnp.int32)

    h1 = _hist_p1(x, zeros16)
    b1, k2 = _pick_bin(h1, 4096, _K)
    h2 = _hist_p2(x, zeros16 + b1)
    b2, k3 = _pick_bin(h2, 4096, k2)
    h3 = _hist_p3(x, zeros16 + ((b1 << 12) | b2))
    b3, _ = _pick_bin(h3, 256, k3)

    t_key = ((b1 << 20) | (b2 << 8) | b3).astype(jnp.int32).reshape(1, 1)

    block_rows = 8
    grid = _ROWS // block_rows
    out = pl.pallas_call(
        _mask_body,
        grid=(grid,),
        in_specs=[
            pl.BlockSpec(memory_space=pltpu.SMEM),
            pl.BlockSpec((block_rows, _COLS), lambda i: (i, 0)),
        ],
        out_specs=pl.BlockSpec((block_rows, _COLS), lambda i: (i, 0)),
        out_shape=jax.ShapeDtypeStruct((_ROWS, _COLS), jnp.float32),
    )(t_key, x)
    return out


# speedup vs baseline: 4.1307x; 1.0863x over previous
import functools

import jax
import jax.numpy as jnp
from jax import lax
from jax.experimental import pallas as pl
from jax.experimental.pallas import tpu as pltpu
from jax.experimental.pallas import tpu_sc as plsc

_ROWS, _COLS = 128, 32768
_N = _ROWS * _COLS
_K = int(_N * 0.9)
_MIN32 = -2147483648

_NTILES = 32
_SHARD = _N // _NTILES
_CHUNK = 16384
_NCHUNK = _SHARD // _CHUNK
_LANES = 16


def _sortable_key(v):
    return jnp.where(v >= 0, v ^ _MIN32, ~v)


_ROWS_PER_TILE = _ROWS // _NTILES
_CHUNKS_PER_ROW = _COLS // _CHUNK


def _hist_body(nbins, bucket_shift, prefix_shift, x_hbm, prefix_hbm,
               out_hbm, buf0_v, buf1_v, pbuf_v, hist_v, red_v, sem0, sem1):
    wid = lax.axis_index("c") * 16 + lax.axis_index("s")
    row0 = wid * _ROWS_PER_TILE

    bufs = (buf0_v, buf1_v)
    sems = (sem0, sem1)

    def _chunk_src(ci):
        row = row0 + ci // _CHUNKS_PER_ROW
        col = (ci % _CHUNKS_PER_ROW) * _CHUNK
        return x_hbm.at[row, pl.ds(col, _CHUNK)]

    descs = [None, None]
    descs[0] = pltpu.async_copy(_chunk_src(0), bufs[0], sems[0])

    @plsc.parallel_loop(0, nbins, unroll=8)
    def _(z):
        hist_v[pl.ds(z * _LANES, _LANES)] = jnp.zeros((_LANES,), jnp.int32)

    if prefix_shift is not None:
        pltpu.sync_copy(prefix_hbm, pbuf_v)
        pvec = pbuf_v[...]
    else:
        pvec = None
    liota = lax.iota(jnp.int32, _LANES)
    lane_base = liota * nbins
    ones = jnp.full((_LANES,), 1, jnp.int32)

    for ci in range(_NCHUNK):
        descs[ci % 2].wait()
        if ci + 1 < _NCHUNK:
            nxt = (ci + 1) % 2
            descs[nxt] = pltpu.async_copy(_chunk_src(ci + 1), bufs[nxt], sems[nxt])
        buf = bufs[ci % 2]

        @plsc.parallel_loop(0, _CHUNK // _LANES, unroll=8)
        def _(j):
            key = _sortable_key(
                plsc.bitcast(buf[pl.ds(j * _LANES, _LANES)], jnp.int32))
            bucket = lax.shift_right_logical(key, bucket_shift) & (nbins - 1)
            idx = lane_base + bucket
            if prefix_shift is None:
                plsc.addupdate_scatter(hist_v, [idx], ones)
            else:
                m = lax.shift_right_logical(key, prefix_shift) == pvec
                plsc.addupdate_scatter(hist_v, [idx], ones, mask=m)

    @plsc.parallel_loop(0, nbins // _LANES, unroll=4)
    def _(c):
        acc = hist_v[pl.ds(c * _LANES, _LANES)]
        for l in range(1, _LANES):
            acc = acc + hist_v[pl.ds(l * nbins + c * _LANES, _LANES)]
        red_v[pl.ds(c * _LANES, _LANES)] = acc
    pltpu.sync_copy(red_v, out_hbm.at[wid])


def _make_hist_kernel(nbins, bucket_shift, prefix_shift):
    mesh = plsc.VectorSubcoreMesh(core_axis_name="c", subcore_axis_name="s")
    return functools.partial(
        pl.kernel,
        out_type=jax.ShapeDtypeStruct((_NTILES, nbins), jnp.int32),
        mesh=mesh,
        scratch_types=[
            pltpu.VMEM((_CHUNK,), jnp.float32),
            pltpu.VMEM((_CHUNK,), jnp.float32),
            pltpu.VMEM((_LANES,), jnp.int32),
            pltpu.VMEM((nbins * _LANES,), jnp.int32),
            pltpu.VMEM((nbins,), jnp.int32),
            pltpu.SemaphoreType.DMA,
            pltpu.SemaphoreType.DMA,
        ],
        compiler_params=pltpu.CompilerParams(needs_layout_passes=False),
    )(functools.partial(_hist_body, nbins, bucket_shift, prefix_shift))


_hist_p1 = _make_hist_kernel(4096, 20, None)
_hist_p2 = _make_hist_kernel(4096, 8, 20)
_hist_p3 = _make_hist_kernel(256, 0, 8)


def _pick_bin(tile_hists, nbins, k):
    del nbins
    hist = tile_hists.sum(axis=0)
    suffix = jnp.cumsum(hist[::-1])[::-1]
    b = jnp.sum((suffix >= k).astype(jnp.int32)) - 1
    k_next = k - (suffix[b] - hist[b])
    return b, k_next


def _mask_body(t_ref, x_ref, out_ref):
    ts = t_ref[0, 0] ^ _MIN32
    ubits = jnp.where(ts >= 0, ts, _MIN32 | (~ts))
    t = lax.bitcast_convert_type(ubits, jnp.float32)
    xv = x_ref[...]
    m = (xv >= t).astype(jnp.float32)
    out_ref[...] = xv + (m - xv)


def kernel(x):
    zeros16 = jnp.zeros((_LANES,), jnp.int32)

    h1 = _hist_p1(x, zeros16)
    b1, k2 = _pick_bin(h1, 4096, _K)
    h2 = _hist_p2(x, zeros16 + b1)
    b2, k3 = _pick_bin(h2, 4096, k2)
    h3 = _hist_p3(x, zeros16 + ((b1 << 12) | b2))
    b3, _ = _pick_bin(h3, 256, k3)

    t_key = ((b1 << 20) | (b2 << 8) | b3).astype(jnp.int32).reshape(1, 1)

    block_rows = 16
    grid = _ROWS // block_rows
    out = pl.pallas_call(
        _mask_body,
        grid=(grid,),
        in_specs=[
            pl.BlockSpec(memory_space=pltpu.SMEM),
            pl.BlockSpec((block_rows, _COLS), lambda i: (i, 0)),
        ],
        out_specs=pl.BlockSpec((block_rows, _COLS), lambda i: (i, 0)),
        out_shape=jax.ShapeDtypeStruct((_ROWS, _COLS), jnp.float32),
    )(t_key, x)
    return out


# speedup vs baseline: 4.1682x; 1.0091x over previous
import functools

import jax
import jax.numpy as jnp
from jax import lax
from jax.experimental import pallas as pl
from jax.experimental.pallas import tpu as pltpu
from jax.experimental.pallas import tpu_sc as plsc

_ROWS, _COLS = 128, 32768
_N = _ROWS * _COLS
_K = int(_N * 0.9)
_MIN32 = -2147483648

_NTILES = 32
_SHARD = _N // _NTILES
_CHUNK = 16384
_NCHUNK = _SHARD // _CHUNK
_LANES = 16


def _sortable_key(v):
    return jnp.where(v >= 0, v ^ _MIN32, ~v)


_ROWS_PER_TILE = _ROWS // _NTILES
_CHUNKS_PER_ROW = _COLS // _CHUNK


def _hist_body(nbins, bucket_shift, prefix_shift, x_hbm, prefix_hbm,
               out_hbm, buf0_v, buf1_v, pbuf_v, hist_v, red_v, sem0, sem1):
    wid = lax.axis_index("c") * 16 + lax.axis_index("s")
    row0 = wid * _ROWS_PER_TILE

    bufs = (buf0_v, buf1_v)
    sems = (sem0, sem1)

    def _chunk_src(ci):
        row = row0 + ci // _CHUNKS_PER_ROW
        col = (ci % _CHUNKS_PER_ROW) * _CHUNK
        return x_hbm.at[row, pl.ds(col, _CHUNK)]

    descs = [None, None]
    descs[0] = pltpu.async_copy(_chunk_src(0), bufs[0], sems[0])

    @plsc.parallel_loop(0, nbins, unroll=8)
    def _(z):
        hist_v[pl.ds(z * _LANES, _LANES)] = jnp.zeros((_LANES,), jnp.int32)

    if prefix_shift is not None:
        pltpu.sync_copy(prefix_hbm, pbuf_v)
        pvec = pbuf_v[...]
    else:
        pvec = None
    liota = lax.iota(jnp.int32, _LANES)
    lane_base = liota * nbins
    ones = jnp.full((_LANES,), 1, jnp.int32)

    for ci in range(_NCHUNK):
        descs[ci % 2].wait()
        if ci + 1 < _NCHUNK:
            nxt = (ci + 1) % 2
            descs[nxt] = pltpu.async_copy(_chunk_src(ci + 1), bufs[nxt], sems[nxt])
        buf = bufs[ci % 2]

        @plsc.parallel_loop(0, _CHUNK // _LANES, unroll=8)
        def _(j):
            key = _sortable_key(
                plsc.bitcast(buf[pl.ds(j * _LANES, _LANES)], jnp.int32))
            bucket = lax.shift_right_logical(key, bucket_shift) & (nbins - 1)
            idx = lane_base + bucket
            if prefix_shift is None:
                plsc.addupdate_scatter(hist_v, [idx], ones)
            else:
                m = lax.shift_right_logical(key, prefix_shift) == pvec
                plsc.addupdate_scatter(hist_v, [idx], ones, mask=m)

    @plsc.parallel_loop(0, nbins // _LANES, unroll=4)
    def _(c):
        acc = hist_v[pl.ds(c * _LANES, _LANES)]
        for l in range(1, _LANES):
            acc = acc + hist_v[pl.ds(l * nbins + c * _LANES, _LANES)]
        red_v[pl.ds(c * _LANES, _LANES)] = acc
    pltpu.sync_copy(red_v, out_hbm.at[wid])


def _make_hist_kernel(nbins, bucket_shift, prefix_shift):
    mesh = plsc.VectorSubcoreMesh(core_axis_name="c", subcore_axis_name="s")
    return functools.partial(
        pl.kernel,
        out_type=jax.ShapeDtypeStruct((_NTILES, nbins), jnp.int32),
        mesh=mesh,
        scratch_types=[
            pltpu.VMEM((_CHUNK,), jnp.float32),
            pltpu.VMEM((_CHUNK,), jnp.float32),
            pltpu.VMEM((_LANES,), jnp.int32),
            pltpu.VMEM((nbins * _LANES,), jnp.int32),
            pltpu.VMEM((nbins,), jnp.int32),
            pltpu.SemaphoreType.DMA,
            pltpu.SemaphoreType.DMA,
        ],
        compiler_params=pltpu.CompilerParams(needs_layout_passes=False),
    )(functools.partial(_hist_body, nbins, bucket_shift, prefix_shift))


_hist_p1 = _make_hist_kernel(4096, 20, None)
_hist_p2 = _make_hist_kernel(4096, 8, 20)
_hist_p3 = _make_hist_kernel(256, 0, 8)


def _pick_bin(tile_hists, nbins, k):
    del nbins
    hist = tile_hists.sum(axis=0)
    suffix = jnp.cumsum(hist[::-1])[::-1]
    b = jnp.sum((suffix >= k).astype(jnp.int32)) - 1
    k_next = k - (suffix[b] - hist[b])
    return b, k_next


def _mask_body(t_ref, x_ref, out_ref):
    ts = t_ref[0, 0] ^ _MIN32
    ubits = jnp.where(ts >= 0, ts, _MIN32 | (~ts))
    t = lax.bitcast_convert_type(ubits, jnp.float32)
    xv = x_ref[...]
    m = (xv >= t).astype(jnp.float32)
    out_ref[...] = xv + (m - xv)


def kernel(x):
    zeros16 = jnp.zeros((_LANES,), jnp.int32)

    h1 = _hist_p1(x, zeros16)
    b1, k2 = _pick_bin(h1, 4096, _K)
    h2 = _hist_p2(x, zeros16 + b1)
    b2, k3 = _pick_bin(h2, 4096, k2)
    h3 = _hist_p3(x, zeros16 + ((b1 << 12) | b2))
    b3, _ = _pick_bin(h3, 256, k3)

    t_key = ((b1 << 20) | (b2 << 8) | b3).astype(jnp.int32).reshape(1, 1)

    block_rows = 32
    grid = _ROWS // block_rows
    out = pl.pallas_call(
        _mask_body,
        grid=(grid,),
        in_specs=[
            pl.BlockSpec(memory_space=pltpu.SMEM),
            pl.BlockSpec((block_rows, _COLS), lambda i: (i, 0)),
        ],
        out_specs=pl.BlockSpec((block_rows, _COLS), lambda i: (i, 0)),
        out_shape=jax.ShapeDtypeStruct((_ROWS, _COLS), jnp.float32),
    )(t_key, x)
    return out
